# baseline (speedup 1.0000x reference)
import functools

import jax
import jax.numpy as jnp
from jax.experimental import pallas as pl
from jax.experimental.pallas import tpu as pltpu

_BC = 384
_BRA = 400


def _sort_key(f32val):
    b = jax.lax.bitcast_convert_type(f32val, jnp.int32)
    return b ^ (jax.lax.shift_right_arithmetic(b, 31) & jnp.int32(0x7FFFFFFF))


def _fused_kernel(a_ref, atail_ref, x_ref, xtail_ref, w_ref, b_ref, out_ref,
                  sup_ref, srow_ref, scol_ref, sel_ref, *, n, k, nc, nra):
    c = pl.program_id(0)
    ncm = nc - 1

    @pl.when(c == 0)
    def _():
        for j in range(ncm):
            xj = x_ref[j * _BC:(j + 1) * _BC, :]
            sup_ref[j:j + 1, :] = jax.lax.dot_general(
                w_ref[...], xj, (((0,), (1,)), ((), ())),
                preferred_element_type=jnp.float32)
        col = ncm * _BC + jax.lax.broadcasted_iota(jnp.int32, (1, _BC), 1)
        sup_t = jax.lax.dot_general(
            w_ref[...], xtail_ref[...], (((0,), (1,)), ((), ())),
            preferred_element_type=jnp.float32)
        sup_ref[ncm:ncm + 1, :] = jnp.where(col < n, sup_t, 0.0)

    @pl.when(c < nra)
    def _():
        acc = jax.lax.dot_general(
            sup_ref[0:1, :], a_ref[:, 0:_BC], (((1,), (1,)), ((), ())),
            preferred_element_type=jnp.float32)
        for j in range(1, ncm):
            acc += jax.lax.dot_general(
                sup_ref[j:j + 1, :], a_ref[:, j * _BC:(j + 1) * _BC],
                (((1,), (1,)), ((), ())),
                preferred_element_type=jnp.float32)
        tcol = ncm * _BC + jax.lax.broadcasted_iota(jnp.int32, (_BRA, 128), 1)
        at = jnp.where(tcol < n, atail_ref[pl.ds(c * _BRA, _BRA), :], 0.0)
        acc += jax.lax.dot_general(
            sup_ref[ncm:ncm + 1, 0:128], at, (((1,), (1,)), ((), ())),
            preferred_element_type=jnp.float32)

        score = jnp.tanh(acc + b_ref[0, 0])
        srow_ref[pl.ds(c, 1), :] = score
        scol_ref[pl.ds(c * _BRA, _BRA), :] = jnp.reshape(score, (_BRA, 1))

    @pl.when(c == nra - 1)
    def _():
        key = _sort_key(srow_ref[...])
        rows = jax.lax.broadcasted_iota(jnp.int32, srow_ref.shape, 0)
        key = jnp.where(rows < nra, key, jnp.int32(-2147483648))
        idx = (rows * _BRA
               + jax.lax.broadcasted_iota(jnp.int32, srow_ref.shape, 1))

        npos = jnp.sum((key >= 0).astype(jnp.int32))
        cand0 = jnp.where(npos >= k, jnp.int32(0), jnp.int32(-2147483648))

        def vbody(i, cand):
            test = cand | (jnp.int32(1) << (30 - i))
            cnt = jnp.sum((key >= test).astype(jnp.int32))
            return jnp.where(cnt >= k, test, cand)

        tkey = jax.lax.fori_loop(0, 31, vbody, cand0)

        eq = key == tkey
        cgt = jnp.sum((key > tkey).astype(jnp.int32))
        need = k - cgt

        def ibody(i, lohi):
            lo, hi = lohi
            mid = (lo + hi) // 2
            cnt = jnp.sum((eq & (idx < mid)).astype(jnp.int32))
            found = cnt >= need
            return (jnp.where(found, lo, mid), jnp.where(found, mid, hi))

        lo, hi = jax.lax.fori_loop(
            0, 14, ibody, (jnp.int32(0), jnp.int32(n)))
        sel_ref[0] = tkey
        sel_ref[1] = jnp.where(need > 0, hi, jnp.int32(0))

    @pl.when(c >= nra)
    def _():
        r = c - nra
        tkey = sel_ref[0]
        cutoff = sel_ref[1]
        sc = scol_ref[pl.ds(r * _BRA, _BRA), :]
        kc = _sort_key(sc)
        ridx = r * _BRA + jax.lax.broadcasted_iota(jnp.int32, (_BRA, 1), 0)
        keep = (kc > tkey) | ((kc == tkey) & (ridx < cutoff))
        coeff = jnp.where(keep, sc, 0.0)
        out_ref[...] = (x_ref[pl.ds(r * _BRA, _BRA), :] * coeff)[None]


def kernel(adjacency, input_feature, weight, bias):
    n, d = input_feature.shape
    k = max(int(0.5 * n), 1)
    nc = (n + _BC - 1) // _BC
    nra = n // _BRA
    nrp = (nra + 7) // 8 * 8

    hidden = pl.pallas_call(
        functools.partial(_fused_kernel, n=n, k=k, nc=nc, nra=nra),
        grid=(2 * nra,),
        in_specs=[
            pl.BlockSpec((_BRA, n), lambda c: (jnp.clip(c, 0, nra - 1), 0)),
            pl.BlockSpec((n, 128), lambda c: (0, (nc - 1) * _BC // 128)),
            pl.BlockSpec((n, d), lambda c: (0, 0)),
            pl.BlockSpec((_BC, d), lambda c: (nc - 1, 0)),
            pl.BlockSpec((d, 1), lambda c: (0, 0)),
            pl.BlockSpec((1, 1), lambda c: (0, 0)),
        ],
        out_specs=pl.BlockSpec(
            (1, _BRA, d), lambda c: (0, jnp.maximum(c - nra, 0), 0)),
        out_shape=jax.ShapeDtypeStruct((1, n, d), jnp.float32),
        scratch_shapes=[
            pltpu.VMEM((32, _BC), jnp.float32),
            pltpu.VMEM((nrp, _BRA), jnp.float32),
            pltpu.VMEM((n, 1), jnp.float32),
            pltpu.SMEM((2,), jnp.int32),
        ],
        compiler_params=pltpu.CompilerParams(
            dimension_semantics=("arbitrary",),
            vmem_limit_bytes=64 * 1024 * 1024),
    )(adjacency, adjacency, input_feature, input_feature,
      weight, bias.reshape(1, 1))

    return hidden


# speedup vs baseline: 1.0391x; 1.0391x over previous
import functools

import jax
import jax.numpy as jnp
from jax.experimental import pallas as pl
from jax.experimental.pallas import tpu as pltpu

_BC = 384
_BR = 1000


def _sort_key(f32val):
    b = jax.lax.bitcast_convert_type(f32val, jnp.int32)
    return b ^ (jax.lax.shift_right_arithmetic(b, 31) & jnp.int32(0x7FFFFFFF))


def _fused_kernel(at_ref, ab_ref, xc_ref, w_ref, b_ref, out_ref,
                  acc_ref, xres_ref, scol_ref, sel_ref, *, n, k, nc):
    c = pl.program_id(0)
    n2 = n // 2

    @pl.when(c == 0)
    def _():
        acc_ref[...] = jnp.zeros_like(acc_ref)

    @pl.when(c < nc)
    def _():
        col = c * _BC + jax.lax.broadcasted_iota(jnp.int32, (1, _BC), 1)
        col_valid = col < n
        x = xc_ref[...]
        xres_ref[pl.ds(c * _BC, _BC), :] = x
        support = jax.lax.dot_general(
            w_ref[...], x, (((0,), (1,)), ((), ())),
            preferred_element_type=jnp.float32)
        support = jnp.where(col_valid, support, 0.0)

        @pl.when(c < nc - 1)
        def _():
            acc_ref[0:1, :] += jax.lax.dot_general(
                support, at_ref[...], (((1,), (1,)), ((), ())),
                preferred_element_type=jnp.float32)
            acc_ref[1:2, :] += jax.lax.dot_general(
                support, ab_ref[...], (((1,), (1,)), ((), ())),
                preferred_element_type=jnp.float32)

        @pl.when(c == nc - 1)
        def _():
            at = jnp.where(col_valid, at_ref[...], 0.0)
            ab = jnp.where(col_valid, ab_ref[...], 0.0)
            acc_ref[0:1, :] += jax.lax.dot_general(
                support, at, (((1,), (1,)), ((), ())),
                preferred_element_type=jnp.float32)
            acc_ref[1:2, :] += jax.lax.dot_general(
                support, ab, (((1,), (1,)), ((), ())),
                preferred_element_type=jnp.float32)

            score = jnp.tanh(acc_ref[...] + b_ref[0, 0])
            scol_ref[pl.ds(0, n2), :] = jnp.reshape(score[0:1, :], (n2, 1))
            scol_ref[pl.ds(n2, n2), :] = jnp.reshape(score[1:2, :], (n2, 1))

            key = _sort_key(score)
            idx = (jax.lax.broadcasted_iota(jnp.int32, (2, n2), 0) * n2
                   + jax.lax.broadcasted_iota(jnp.int32, (2, n2), 1))

            npos = jnp.sum((key >= 0).astype(jnp.int32), keepdims=True)
            cand0 = jnp.where(npos >= k, jnp.int32(0), jnp.int32(-2147483648))

            def vbody(i, cand):
                test = cand | (jnp.int32(1) << (30 - i))
                cnt = jnp.sum((key >= test).astype(jnp.int32), keepdims=True)
                return jnp.where(cnt >= k, test, cand)

            tkey = jax.lax.fori_loop(0, 31, vbody, cand0)

            eq = key == tkey
            cgt = jnp.sum((key > tkey).astype(jnp.int32), keepdims=True)
            need = k - cgt

            def ibody(i, lohi):
                lo, hi = lohi
                mid = (lo + hi) // 2
                cnt = jnp.sum((eq & (idx < mid)).astype(jnp.int32),
                              keepdims=True)
                found = cnt >= need
                return (jnp.where(found, lo, mid), jnp.where(found, mid, hi))

            lo, hi = jax.lax.fori_loop(
                0, 14, ibody,
                (jnp.zeros((1, 1), jnp.int32), jnp.full((1, 1), n, jnp.int32)))
            sel_ref[0] = tkey[0, 0]
            sel_ref[1] = jnp.where(need > 0, hi, 0)[0, 0]

    @pl.when(c >= nc)
    def _():
        r = c - nc
        tkey = sel_ref[0]
        cutoff = sel_ref[1]
        sc = scol_ref[pl.ds(r * _BR, _BR), :]
        kc = _sort_key(sc)
        ridx = r * _BR + jax.lax.broadcasted_iota(jnp.int32, (_BR, 1), 0)
        keep = (kc > tkey) | ((kc == tkey) & (ridx < cutoff))
        coeff = jnp.where(keep, sc, 0.0)
        out_ref[...] = (xres_ref[pl.ds(r * _BR, _BR), :] * coeff)[None]


def kernel(adjacency, input_feature, weight, bias):
    n, d = input_feature.shape
    k = max(int(0.5 * n), 1)
    nc = (n + _BC - 1) // _BC
    nr = n // _BR

    hidden = pl.pallas_call(
        functools.partial(_fused_kernel, n=n, k=k, nc=nc),
        grid=(nc + nr,),
        in_specs=[
            pl.BlockSpec((n // 2, _BC), lambda c: (0, jnp.minimum(c, nc - 1))),
            pl.BlockSpec((n // 2, _BC), lambda c: (1, jnp.minimum(c, nc - 1))),
            pl.BlockSpec((_BC, d), lambda c: (jnp.minimum(c, nc - 1), 0)),
            pl.BlockSpec((d, 1), lambda c: (0, 0)),
            pl.BlockSpec((1, 1), lambda c: (0, 0)),
        ],
        out_specs=pl.BlockSpec((1, _BR, d), lambda c: (0, jnp.maximum(c - nc, 0), 0)),
        out_shape=jax.ShapeDtypeStruct((1, n, d), jnp.float32),
        scratch_shapes=[
            pltpu.VMEM((2, n // 2), jnp.float32),
            pltpu.VMEM((nc * _BC, d), jnp.float32),
            pltpu.VMEM((n, 1), jnp.float32),
            pltpu.SMEM((2,), jnp.int32),
        ],
        compiler_params=pltpu.CompilerParams(
            dimension_semantics=("arbitrary",),
            vmem_limit_bytes=64 * 1024 * 1024),
    )(adjacency, adjacency, input_feature, weight, bias.reshape(1, 1))

    return hidden
